# baseline (speedup 1.0000x reference)
import functools

import jax
import jax.numpy as jnp
import numpy as np
from jax import lax
from jax.experimental import pallas as pl
from jax.experimental.pallas import tpu as pltpu
from jax.experimental.pallas import tpu_sc as plsc

_L = 1024
_B = 64
_D = 768
_V = 8192
_NW = 32
_RPW = (_B * _L) // _NW
_NKEEP = _RPW // 4
_NMASK = _RPW - _NKEEP
_KC = 32
_NKC = _NKEEP // _KC
_NB = 4
_TCA = 128
_NTCA = _NMASK // _TCA


def _threefry2x32_np(k1, k2, x1, x2):
    m = np.uint64(0xFFFFFFFF)

    def rotl(x, d):
        return ((x << np.uint64(d)) | (x >> np.uint64(32 - d))) & m

    x1 = x1.astype(np.uint64)
    x2 = x2.astype(np.uint64)
    ks = [np.uint64(k1), np.uint64(k2),
          np.uint64(k1) ^ np.uint64(k2) ^ np.uint64(0x1BD11BDA)]
    rot = [[13, 15, 26, 6], [17, 29, 16, 24]]
    x1 = (x1 + ks[0]) & m
    x2 = (x2 + ks[1]) & m
    for r in range(5):
        for d in rot[r % 2]:
            x1 = (x1 + x2) & m
            x2 = rotl(x2, d)
            x2 = x1 ^ x2
        x1 = (x1 + ks[(r + 1) % 3]) & m
        x2 = (x2 + ks[(r + 2) % 3] + np.uint64(r + 1)) & m
    return x1.astype(np.uint32), x2.astype(np.uint32)


def _uniform_np(seed, n):
    i = np.arange(n, dtype=np.uint64)
    hi = (i >> np.uint64(32)).astype(np.uint32)
    lo = (i & np.uint64(0xFFFFFFFF)).astype(np.uint32)
    o1, o2 = _threefry2x32_np(0, np.uint32(seed), hi, lo)
    bits = o1 ^ o2
    return (((bits >> np.uint32(9)) | np.uint32(0x3F800000)).view(np.float32)
            - np.float32(1.0))


def _mask_constants():
    noise = _uniform_np(42, _B * _L).reshape(_B, _L)
    order = np.argsort(noise, axis=1, kind="stable")
    ranks = np.argsort(order, axis=1, kind="stable")
    mask = ranks >= (_L // 4)
    ids = np.argsort(mask.reshape(_NW, _RPW), axis=1, kind="stable")
    off = (np.arange(_NW, dtype=np.int64) * _RPW)[:, None]
    kidx = (ids[:, :_NKEEP] + off).astype(np.int32).reshape(_NW, _NKC, _KC)
    midx = (ids[:, _NKEEP:] + off).astype(np.int32).reshape(_NW, _NTCA, _TCA)
    return mask, kidx, midx


_MASK_NP, _KIDX_NP, _MIDX_NP = _mask_constants()


def _proj_table_kernel(tab_ref, w_ref, b_ref, out_ref):
    out_ref[...] = (
        jnp.dot(tab_ref[...], w_ref[...], preferred_element_type=jnp.float32)
        + b_ref[...]
    )


def _build_proj_table(table, w, b):
    blk = 1024
    return pl.pallas_call(
        _proj_table_kernel,
        grid=(_V // blk,),
        in_specs=[
            pl.BlockSpec((blk, 32), lambda i: (i, 0)),
            pl.BlockSpec((32, _D), lambda i: (0, 0)),
            pl.BlockSpec((1, _D), lambda i: (0, 0)),
        ],
        out_specs=pl.BlockSpec((blk, _D), lambda i: (i, 0)),
        out_shape=jax.ShapeDtypeStruct((_V, _D), jnp.float32),
    )(table, w, b)


def _sc_token_fill(tokrep, midx_a):
    mesh = plsc.VectorSubcoreMesh(core_axis_name="c", subcore_axis_name="s")

    @functools.partial(
        pl.kernel,
        mesh=mesh,
        out_type=jax.ShapeDtypeStruct((_B * _L, _D), jnp.float32),
        scratch_types=[
            pltpu.VMEM((_TCA, _D), jnp.float32),
            pltpu.VMEM((_NTCA, _TCA), jnp.int32),
            pltpu.SemaphoreType.DMA,
        ],
    )
    def k(tok_hbm, midx_hbm, out_hbm, tokbuf, midx_v, st):
        wid = lax.axis_index("s") * 2 + lax.axis_index("c")
        pltpu.sync_copy(tok_hbm, tokbuf)
        pltpu.sync_copy(midx_hbm.at[wid], midx_v)
        th = []
        for j in range(_NTCA):
            th.append(pltpu.async_copy(tokbuf, out_hbm.at[midx_v.at[j]], st))
            if j >= 4:
                th[j - 4].wait()
        for j in range(max(0, _NTCA - 4), _NTCA):
            th[j].wait()

    return k(tokrep, midx_a)


def _sc_kept_scatter(projT, x_flat, kidx_g, out_ref):
    mesh = plsc.VectorSubcoreMesh(core_axis_name="c", subcore_axis_name="s")

    @functools.partial(
        pl.kernel,
        mesh=mesh,
        scratch_types=[
            [pltpu.VMEM((_KC, _D), jnp.float32) for _ in range(_NB)],
            pltpu.VMEM((_NKEEP,), jnp.int32),
            pltpu.VMEM((_NKEEP,), jnp.int32),
            pltpu.VMEM((_NKC, _KC), jnp.int32),
            pltpu.SemaphoreType.DMA,
            [pltpu.SemaphoreType.DMA for _ in range(_NB)],
            [pltpu.SemaphoreType.DMA for _ in range(_NB)],
        ],
    )
    def k(tab_hbm, x_hbm, kgf_hbm, kg_hbm, out_hbm, rows, iv, kgf, kg_v,
          isem, gsem, ssem):
        wid = lax.axis_index("s") * 2 + lax.axis_index("c")
        pltpu.sync_copy(kg_hbm.at[wid], kg_v)
        pltpu.sync_copy(kgf_hbm.at[wid], kgf)

        ih = []
        for q in range(_NKEEP // 128):
            ih.append(pltpu.async_copy(
                x_hbm.at[kgf.at[pl.ds(q * 128, 128)]],
                iv.at[pl.ds(q * 128, 128)], isem))

        waited = set()

        def row_gather(i):
            q = (i * _KC) // 128
            if q not in waited:
                ih[q].wait()
                waited.add(q)
            return pltpu.async_copy(
                tab_hbm.at[iv.at[pl.ds(i * _KC, _KC)]], rows[i % _NB],
                gsem[i % _NB])

        gh = {}
        sh = {}
        for i in range(2):
            gh[i] = row_gather(i)
        for i in range(_NKC):
            gh[i].wait()
            sh[i] = pltpu.async_copy(
                rows[i % _NB], out_hbm.at[kg_v.at[i]], ssem[i % _NB])
            j = i + 2
            if j < _NKC:
                if j >= _NB:
                    sh[j - _NB].wait()
                gh[j] = row_gather(j)
        for i in range(_NKC - _NB, _NKC):
            sh[i].wait()

    return k(projT, x_flat, kidx_g.reshape(_NW, _NKEEP), kidx_g, out_ref)


def kernel(x, patch_embed_weight, proj_w, proj_b, mask_token):
    mask = jnp.asarray(_MASK_NP)
    midx = jnp.asarray(_MIDX_NP)
    kidx_g = jnp.asarray(_KIDX_NP)

    x_flat = x.reshape(-1).astype(jnp.int32)
    tokrep = jnp.broadcast_to(
        mask_token.reshape(1, _D).astype(jnp.float32), (_TCA, _D))

    filled = _sc_token_fill(tokrep, midx)
    projT = _build_proj_table(
        patch_embed_weight.astype(jnp.float32),
        proj_w.astype(jnp.float32),
        proj_b.reshape(1, _D).astype(jnp.float32),
    )
    out_ref = jax.new_ref(filled)
    _sc_kept_scatter(projT, x_flat, kidx_g, out_ref)
    return out_ref[...].reshape(_B, _L, _D), mask


# speedup vs baseline: 1.0160x; 1.0160x over previous
import functools

import jax
import jax.numpy as jnp
import numpy as np
from jax import lax
from jax.experimental import pallas as pl
from jax.experimental.pallas import tpu as pltpu
from jax.experimental.pallas import tpu_sc as plsc

_L = 1024
_B = 64
_D = 768
_V = 8192
_NW = 32
_RPW = (_B * _L) // _NW
_NKEEP = _RPW // 4
_NMASK = _RPW - _NKEEP
_KC = 32
_NKC = _NKEEP // _KC
_NB = 4
_TCA = 128
_NTCA = _NMASK // _TCA


def _threefry2x32_np(k1, k2, x1, x2):
    m = np.uint64(0xFFFFFFFF)

    def rotl(x, d):
        return ((x << np.uint64(d)) | (x >> np.uint64(32 - d))) & m

    x1 = x1.astype(np.uint64)
    x2 = x2.astype(np.uint64)
    ks = [np.uint64(k1), np.uint64(k2),
          np.uint64(k1) ^ np.uint64(k2) ^ np.uint64(0x1BD11BDA)]
    rot = [[13, 15, 26, 6], [17, 29, 16, 24]]
    x1 = (x1 + ks[0]) & m
    x2 = (x2 + ks[1]) & m
    for r in range(5):
        for d in rot[r % 2]:
            x1 = (x1 + x2) & m
            x2 = rotl(x2, d)
            x2 = x1 ^ x2
        x1 = (x1 + ks[(r + 1) % 3]) & m
        x2 = (x2 + ks[(r + 2) % 3] + np.uint64(r + 1)) & m
    return x1.astype(np.uint32), x2.astype(np.uint32)


def _uniform_np(seed, n):
    i = np.arange(n, dtype=np.uint64)
    hi = (i >> np.uint64(32)).astype(np.uint32)
    lo = (i & np.uint64(0xFFFFFFFF)).astype(np.uint32)
    o1, o2 = _threefry2x32_np(0, np.uint32(seed), hi, lo)
    bits = o1 ^ o2
    return (((bits >> np.uint32(9)) | np.uint32(0x3F800000)).view(np.float32)
            - np.float32(1.0))


def _mask_constants():
    noise = _uniform_np(42, _B * _L).reshape(_B, _L)
    order = np.argsort(noise, axis=1, kind="stable")
    ranks = np.argsort(order, axis=1, kind="stable")
    mask = ranks >= (_L // 4)
    ids = np.argsort(mask.reshape(_NW, _RPW), axis=1, kind="stable")
    off = (np.arange(_NW, dtype=np.int64) * _RPW)[:, None]
    kidx = (ids[:, :_NKEEP] + off).astype(np.int32).reshape(_NW, _NKC, _KC)
    midx = (ids[:, _NKEEP:] + off).astype(np.int32).reshape(_NW, _NTCA, _TCA)
    return mask, kidx, midx


_MASK_NP, _KIDX_NP, _MIDX_NP = _mask_constants()


def _proj_table_kernel(tab_ref, w_ref, b_ref, out_ref):
    out_ref[...] = (
        jnp.dot(tab_ref[...], w_ref[...], preferred_element_type=jnp.float32)
        + b_ref[...]
    )


def _build_proj_table(table, w, b):
    blk = 1024
    return pl.pallas_call(
        _proj_table_kernel,
        grid=(_V // blk,),
        in_specs=[
            pl.BlockSpec((blk, 32), lambda i: (i, 0)),
            pl.BlockSpec((32, _D), lambda i: (0, 0)),
            pl.BlockSpec((1, _D), lambda i: (0, 0)),
        ],
        out_specs=pl.BlockSpec((blk, _D), lambda i: (i, 0)),
        out_shape=jax.ShapeDtypeStruct((_V, _D), jnp.float32),
    )(table, w, b)


def _sc_token_fill(tokrep, midx_a):
    mesh = plsc.VectorSubcoreMesh(core_axis_name="c", subcore_axis_name="s")

    @functools.partial(
        pl.kernel,
        mesh=mesh,
        out_type=jax.ShapeDtypeStruct((_B * _L, _D), jnp.float32),
        scratch_types=[
            pltpu.VMEM((_TCA, _D), jnp.float32),
            pltpu.VMEM((_NTCA, _TCA), jnp.int32),
            pltpu.SemaphoreType.DMA,
        ],
    )
    def k(tok_hbm, midx_hbm, out_hbm, tokbuf, midx_v, st):
        wid = lax.axis_index("s") * 2 + lax.axis_index("c")
        pltpu.sync_copy(tok_hbm, tokbuf)
        pltpu.sync_copy(midx_hbm.at[wid], midx_v)
        th = []
        for j in range(_NTCA):
            th.append(pltpu.async_copy(tokbuf, out_hbm.at[midx_v.at[j]], st))
            if j >= 4:
                th[j - 4].wait()
        for j in range(max(0, _NTCA - 4), _NTCA):
            th[j].wait()

    return k(tokrep, midx_a)


def _sc_kept_scatter(projT, x_flat, kidx_g, out_ref):
    mesh = plsc.VectorSubcoreMesh(core_axis_name="c", subcore_axis_name="s")

    @functools.partial(
        pl.kernel,
        mesh=mesh,
        scratch_types=[
            [pltpu.VMEM((_KC, _D), jnp.float32) for _ in range(_NB)],
            [pltpu.VMEM((_KC,), jnp.int32) for _ in range(_NB)],
            pltpu.VMEM((_NKC, _KC), jnp.int32),
            [pltpu.SemaphoreType.DMA for _ in range(_NB)],
            [pltpu.SemaphoreType.DMA for _ in range(_NB)],
            [pltpu.SemaphoreType.DMA for _ in range(_NB)],
        ],
    )
    def k(tab_hbm, x_hbm, kg_hbm, out_hbm, rows, ivs, kg_v, isem, gsem, ssem):
        wid = lax.axis_index("s") * 2 + lax.axis_index("c")
        pltpu.sync_copy(kg_hbm.at[wid], kg_v)

        def idx_gather(i):
            return pltpu.async_copy(
                x_hbm.at[kg_v.at[i]], ivs[i % _NB], isem[i % _NB])

        def row_gather(i):
            return pltpu.async_copy(
                tab_hbm.at[ivs[i % _NB]], rows[i % _NB], gsem[i % _NB])

        ih = {i: idx_gather(i) for i in range(_NB)}
        gh = {}
        sh = {}
        for i in range(2):
            ih[i].wait()
            gh[i] = row_gather(i)
        for i in range(_NKC):
            gh[i].wait()
            sh[i] = pltpu.async_copy(
                rows[i % _NB], out_hbm.at[kg_v.at[i]], ssem[i % _NB])
            if i + _NB < _NKC:
                ih[i + _NB] = idx_gather(i + _NB)
            j = i + 2
            if j < _NKC:
                if j >= _NB:
                    sh[j - _NB].wait()
                ih[j].wait()
                gh[j] = row_gather(j)
        for i in range(_NKC - _NB, _NKC):
            sh[i].wait()

    return k(projT, x_flat, kidx_g, out_ref)


def kernel(x, patch_embed_weight, proj_w, proj_b, mask_token):
    mask = jnp.asarray(_MASK_NP)
    midx = jnp.asarray(_MIDX_NP)
    kidx_g = jnp.asarray(_KIDX_NP)

    x_flat = x.reshape(-1).astype(jnp.int32)
    tokrep = jnp.broadcast_to(
        mask_token.reshape(1, _D).astype(jnp.float32), (_TCA, _D))

    filled = _sc_token_fill(tokrep, midx)
    projT = _build_proj_table(
        patch_embed_weight.astype(jnp.float32),
        proj_w.astype(jnp.float32),
        proj_b.reshape(1, _D).astype(jnp.float32),
    )
    out_ref = jax.new_ref(filled)
    _sc_kept_scatter(projT, x_flat, kidx_g, out_ref)
    return out_ref[...].reshape(_B, _L, _D), mask


# speedup vs baseline: 1.0425x; 1.0261x over previous
import functools

import jax
import jax.numpy as jnp
import numpy as np
from jax import lax
from jax.experimental import pallas as pl
from jax.experimental.pallas import tpu as pltpu
from jax.experimental.pallas import tpu_sc as plsc

_L = 1024
_B = 64
_D = 768
_V = 8192
_NW = 32
_RPW = (_B * _L) // _NW
_NKEEP = _RPW // 4
_NMASK = _RPW - _NKEEP
_KC = 32
_NKC = _NKEEP // _KC
_NB = 4
_TCA = 64
_NTCA = _NMASK // _TCA


def _threefry2x32_np(k1, k2, x1, x2):
    m = np.uint64(0xFFFFFFFF)

    def rotl(x, d):
        return ((x << np.uint64(d)) | (x >> np.uint64(32 - d))) & m

    x1 = x1.astype(np.uint64)
    x2 = x2.astype(np.uint64)
    ks = [np.uint64(k1), np.uint64(k2),
          np.uint64(k1) ^ np.uint64(k2) ^ np.uint64(0x1BD11BDA)]
    rot = [[13, 15, 26, 6], [17, 29, 16, 24]]
    x1 = (x1 + ks[0]) & m
    x2 = (x2 + ks[1]) & m
    for r in range(5):
        for d in rot[r % 2]:
            x1 = (x1 + x2) & m
            x2 = rotl(x2, d)
            x2 = x1 ^ x2
        x1 = (x1 + ks[(r + 1) % 3]) & m
        x2 = (x2 + ks[(r + 2) % 3] + np.uint64(r + 1)) & m
    return x1.astype(np.uint32), x2.astype(np.uint32)


def _uniform_np(seed, n):
    i = np.arange(n, dtype=np.uint64)
    hi = (i >> np.uint64(32)).astype(np.uint32)
    lo = (i & np.uint64(0xFFFFFFFF)).astype(np.uint32)
    o1, o2 = _threefry2x32_np(0, np.uint32(seed), hi, lo)
    bits = o1 ^ o2
    return (((bits >> np.uint32(9)) | np.uint32(0x3F800000)).view(np.float32)
            - np.float32(1.0))


def _mask_constants():
    noise = _uniform_np(42, _B * _L).reshape(_B, _L)
    order = np.argsort(noise, axis=1, kind="stable")
    ranks = np.argsort(order, axis=1, kind="stable")
    mask = ranks >= (_L // 4)
    ids = np.argsort(mask.reshape(_NW, _RPW), axis=1, kind="stable")
    off = (np.arange(_NW, dtype=np.int64) * _RPW)[:, None]
    kidx = (ids[:, :_NKEEP] + off).astype(np.int32).reshape(_NW, _NKC, _KC)
    midx = (ids[:, _NKEEP:] + off).astype(np.int32).reshape(_NW, _NTCA, _TCA)
    return mask, kidx, midx


_MASK_NP, _KIDX_NP, _MIDX_NP = _mask_constants()


def _proj_table_kernel(tab_ref, w_ref, b_ref, out_ref):
    out_ref[...] = (
        jnp.dot(tab_ref[...], w_ref[...], preferred_element_type=jnp.float32)
        + b_ref[...]
    )


def _build_proj_table(table, w, b):
    blk = 1024
    return pl.pallas_call(
        _proj_table_kernel,
        grid=(_V // blk,),
        in_specs=[
            pl.BlockSpec((blk, 32), lambda i: (i, 0)),
            pl.BlockSpec((32, _D), lambda i: (0, 0)),
            pl.BlockSpec((1, _D), lambda i: (0, 0)),
        ],
        out_specs=pl.BlockSpec((blk, _D), lambda i: (i, 0)),
        out_shape=jax.ShapeDtypeStruct((_V, _D), jnp.float32),
    )(table, w, b)


def _sc_token_fill(tokrep, midx_a):
    mesh = plsc.VectorSubcoreMesh(core_axis_name="c", subcore_axis_name="s")

    @functools.partial(
        pl.kernel,
        mesh=mesh,
        out_type=jax.ShapeDtypeStruct((_B * _L, _D), jnp.float32),
        scratch_types=[
            pltpu.VMEM((_TCA, _D), jnp.float32),
            pltpu.VMEM((_NTCA, _TCA), jnp.int32),
            pltpu.SemaphoreType.DMA,
        ],
    )
    def k(tok_hbm, midx_hbm, out_hbm, tokbuf, midx_v, st):
        wid = lax.axis_index("s") * 2 + lax.axis_index("c")
        pltpu.sync_copy(tok_hbm, tokbuf)
        pltpu.sync_copy(midx_hbm.at[wid], midx_v)
        th = []
        for j in range(_NTCA):
            th.append(pltpu.async_copy(tokbuf, out_hbm.at[midx_v.at[j]], st))
            if j >= 4:
                th[j - 4].wait()
        for j in range(max(0, _NTCA - 4), _NTCA):
            th[j].wait()

    return k(tokrep, midx_a)


def _sc_kept_scatter(projT, x_flat, kidx_g, out_ref):
    mesh = plsc.VectorSubcoreMesh(core_axis_name="c", subcore_axis_name="s")

    @functools.partial(
        pl.kernel,
        mesh=mesh,
        scratch_types=[
            [pltpu.VMEM((_KC, _D), jnp.float32) for _ in range(_NB)],
            [pltpu.VMEM((_KC,), jnp.int32) for _ in range(_NB)],
            pltpu.VMEM((_NKC, _KC), jnp.int32),
            [pltpu.SemaphoreType.DMA for _ in range(_NB)],
            [pltpu.SemaphoreType.DMA for _ in range(_NB)],
            [pltpu.SemaphoreType.DMA for _ in range(_NB)],
        ],
    )
    def k(tab_hbm, x_hbm, kg_hbm, out_hbm, rows, ivs, kg_v, isem, gsem, ssem):
        wid = lax.axis_index("s") * 2 + lax.axis_index("c")
        pltpu.sync_copy(kg_hbm.at[wid], kg_v)

        def idx_gather(i):
            return pltpu.async_copy(
                x_hbm.at[kg_v.at[i]], ivs[i % _NB], isem[i % _NB])

        def row_gather(i):
            return pltpu.async_copy(
                tab_hbm.at[ivs[i % _NB]], rows[i % _NB], gsem[i % _NB])

        ih = {i: idx_gather(i) for i in range(_NB)}
        gh = {}
        sh = {}
        for i in range(2):
            ih[i].wait()
            gh[i] = row_gather(i)
        for i in range(_NKC):
            gh[i].wait()
            sh[i] = pltpu.async_copy(
                rows[i % _NB], out_hbm.at[kg_v.at[i]], ssem[i % _NB])
            if i + _NB < _NKC:
                ih[i + _NB] = idx_gather(i + _NB)
            j = i + 2
            if j < _NKC:
                if j >= _NB:
                    sh[j - _NB].wait()
                ih[j].wait()
                gh[j] = row_gather(j)
        for i in range(_NKC - _NB, _NKC):
            sh[i].wait()

    return k(projT, x_flat, kidx_g, out_ref)


def kernel(x, patch_embed_weight, proj_w, proj_b, mask_token):
    mask = jnp.asarray(_MASK_NP)
    midx = jnp.asarray(_MIDX_NP)
    kidx_g = jnp.asarray(_KIDX_NP)

    x_flat = x.reshape(-1).astype(jnp.int32)
    tokrep = jnp.broadcast_to(
        mask_token.reshape(1, _D).astype(jnp.float32), (_TCA, _D))

    filled = _sc_token_fill(tokrep, midx)
    projT = _build_proj_table(
        patch_embed_weight.astype(jnp.float32),
        proj_w.astype(jnp.float32),
        proj_b.reshape(1, _D).astype(jnp.float32),
    )
    out_ref = jax.new_ref(filled)
    _sc_kept_scatter(projT, x_flat, kidx_g, out_ref)
    return out_ref[...].reshape(_B, _L, _D), mask


# speedup vs baseline: 1.0566x; 1.0135x over previous
import functools

import jax
import jax.numpy as jnp
import numpy as np
from jax import lax
from jax.experimental import pallas as pl
from jax.experimental.pallas import tpu as pltpu
from jax.experimental.pallas import tpu_sc as plsc

_L = 1024
_B = 64
_D = 768
_V = 8192
_NW = 32
_RPW = (_B * _L) // _NW
_NKEEP = _RPW // 4
_NMASK = _RPW - _NKEEP
_KC = 32
_NKC = _NKEEP // _KC
_NB = 4
_TCA = 32
_NTCA = _NMASK // _TCA


def _threefry2x32_np(k1, k2, x1, x2):
    m = np.uint64(0xFFFFFFFF)

    def rotl(x, d):
        return ((x << np.uint64(d)) | (x >> np.uint64(32 - d))) & m

    x1 = x1.astype(np.uint64)
    x2 = x2.astype(np.uint64)
    ks = [np.uint64(k1), np.uint64(k2),
          np.uint64(k1) ^ np.uint64(k2) ^ np.uint64(0x1BD11BDA)]
    rot = [[13, 15, 26, 6], [17, 29, 16, 24]]
    x1 = (x1 + ks[0]) & m
    x2 = (x2 + ks[1]) & m
    for r in range(5):
        for d in rot[r % 2]:
            x1 = (x1 + x2) & m
            x2 = rotl(x2, d)
            x2 = x1 ^ x2
        x1 = (x1 + ks[(r + 1) % 3]) & m
        x2 = (x2 + ks[(r + 2) % 3] + np.uint64(r + 1)) & m
    return x1.astype(np.uint32), x2.astype(np.uint32)


def _uniform_np(seed, n):
    i = np.arange(n, dtype=np.uint64)
    hi = (i >> np.uint64(32)).astype(np.uint32)
    lo = (i & np.uint64(0xFFFFFFFF)).astype(np.uint32)
    o1, o2 = _threefry2x32_np(0, np.uint32(seed), hi, lo)
    bits = o1 ^ o2
    return (((bits >> np.uint32(9)) | np.uint32(0x3F800000)).view(np.float32)
            - np.float32(1.0))


def _mask_constants():
    noise = _uniform_np(42, _B * _L).reshape(_B, _L)
    order = np.argsort(noise, axis=1, kind="stable")
    ranks = np.argsort(order, axis=1, kind="stable")
    mask = ranks >= (_L // 4)
    ids = np.argsort(mask.reshape(_NW, _RPW), axis=1, kind="stable")
    off = (np.arange(_NW, dtype=np.int64) * _RPW)[:, None]
    kidx = (ids[:, :_NKEEP] + off).astype(np.int32).reshape(_NW, _NKC, _KC)
    midx = (ids[:, _NKEEP:] + off).astype(np.int32).reshape(_NW, _NTCA, _TCA)
    return mask, kidx, midx


_MASK_NP, _KIDX_NP, _MIDX_NP = _mask_constants()


def _proj_table_kernel(tab_ref, w_ref, b_ref, out_ref):
    out_ref[...] = (
        jnp.dot(tab_ref[...], w_ref[...], preferred_element_type=jnp.float32)
        + b_ref[...]
    )


def _build_proj_table(table, w, b):
    blk = 1024
    return pl.pallas_call(
        _proj_table_kernel,
        grid=(_V // blk,),
        in_specs=[
            pl.BlockSpec((blk, 32), lambda i: (i, 0)),
            pl.BlockSpec((32, _D), lambda i: (0, 0)),
            pl.BlockSpec((1, _D), lambda i: (0, 0)),
        ],
        out_specs=pl.BlockSpec((blk, _D), lambda i: (i, 0)),
        out_shape=jax.ShapeDtypeStruct((_V, _D), jnp.float32),
    )(table, w, b)


def _sc_token_fill(tokrep, midx_a):
    mesh = plsc.VectorSubcoreMesh(core_axis_name="c", subcore_axis_name="s")

    @functools.partial(
        pl.kernel,
        mesh=mesh,
        out_type=jax.ShapeDtypeStruct((_B * _L, _D), jnp.float32),
        scratch_types=[
            pltpu.VMEM((_TCA, _D), jnp.float32),
            pltpu.VMEM((_NTCA, _TCA), jnp.int32),
            pltpu.SemaphoreType.DMA,
        ],
    )
    def k(tok_hbm, midx_hbm, out_hbm, tokbuf, midx_v, st):
        wid = lax.axis_index("s") * 2 + lax.axis_index("c")
        pltpu.sync_copy(tok_hbm, tokbuf)
        pltpu.sync_copy(midx_hbm.at[wid], midx_v)
        th = []
        for j in range(_NTCA):
            th.append(pltpu.async_copy(tokbuf, out_hbm.at[midx_v.at[j]], st))
            if j >= 4:
                th[j - 4].wait()
        for j in range(max(0, _NTCA - 4), _NTCA):
            th[j].wait()

    return k(tokrep, midx_a)


def _sc_kept_scatter(projT, x_flat, kidx_g, out_ref):
    mesh = plsc.VectorSubcoreMesh(core_axis_name="c", subcore_axis_name="s")

    @functools.partial(
        pl.kernel,
        mesh=mesh,
        scratch_types=[
            [pltpu.VMEM((_KC, _D), jnp.float32) for _ in range(_NB)],
            [pltpu.VMEM((_KC,), jnp.int32) for _ in range(_NB)],
            pltpu.VMEM((_NKC, _KC), jnp.int32),
            [pltpu.SemaphoreType.DMA for _ in range(_NB)],
            [pltpu.SemaphoreType.DMA for _ in range(_NB)],
            [pltpu.SemaphoreType.DMA for _ in range(_NB)],
        ],
    )
    def k(tab_hbm, x_hbm, kg_hbm, out_hbm, rows, ivs, kg_v, isem, gsem, ssem):
        wid = lax.axis_index("s") * 2 + lax.axis_index("c")
        pltpu.sync_copy(kg_hbm.at[wid], kg_v)

        def idx_gather(i):
            return pltpu.async_copy(
                x_hbm.at[kg_v.at[i]], ivs[i % _NB], isem[i % _NB])

        def row_gather(i):
            return pltpu.async_copy(
                tab_hbm.at[ivs[i % _NB]], rows[i % _NB], gsem[i % _NB])

        ih = {i: idx_gather(i) for i in range(_NB)}
        gh = {}
        sh = {}
        for i in range(2):
            ih[i].wait()
            gh[i] = row_gather(i)
        for i in range(_NKC):
            gh[i].wait()
            sh[i] = pltpu.async_copy(
                rows[i % _NB], out_hbm.at[kg_v.at[i]], ssem[i % _NB])
            if i + _NB < _NKC:
                ih[i + _NB] = idx_gather(i + _NB)
            j = i + 2
            if j < _NKC:
                if j >= _NB:
                    sh[j - _NB].wait()
                ih[j].wait()
                gh[j] = row_gather(j)
        for i in range(_NKC - _NB, _NKC):
            sh[i].wait()

    return k(projT, x_flat, kidx_g, out_ref)


def kernel(x, patch_embed_weight, proj_w, proj_b, mask_token):
    mask = jnp.asarray(_MASK_NP)
    midx = jnp.asarray(_MIDX_NP)
    kidx_g = jnp.asarray(_KIDX_NP)

    x_flat = x.reshape(-1).astype(jnp.int32)
    tokrep = jnp.broadcast_to(
        mask_token.reshape(1, _D).astype(jnp.float32), (_TCA, _D))

    filled = _sc_token_fill(tokrep, midx)
    projT = _build_proj_table(
        patch_embed_weight.astype(jnp.float32),
        proj_w.astype(jnp.float32),
        proj_b.reshape(1, _D).astype(jnp.float32),
    )
    out_ref = jax.new_ref(filled)
    _sc_kept_scatter(projT, x_flat, kidx_g, out_ref)
    return out_ref[...].reshape(_B, _L, _D), mask


# speedup vs baseline: 1.0578x; 1.0011x over previous
import functools

import jax
import jax.numpy as jnp
import numpy as np
from jax import lax
from jax.experimental import pallas as pl
from jax.experimental.pallas import tpu as pltpu
from jax.experimental.pallas import tpu_sc as plsc

_L = 1024
_B = 64
_D = 768
_V = 8192
_NW = 32
_RPW = (_B * _L) // _NW
_NKEEP = _RPW // 4
_NMASK = _RPW - _NKEEP
_KC = 32
_NKC = _NKEEP // _KC
_NB = 4
_TCA = 16
_NTCA = _NMASK // _TCA


def _threefry2x32_np(k1, k2, x1, x2):
    m = np.uint64(0xFFFFFFFF)

    def rotl(x, d):
        return ((x << np.uint64(d)) | (x >> np.uint64(32 - d))) & m

    x1 = x1.astype(np.uint64)
    x2 = x2.astype(np.uint64)
    ks = [np.uint64(k1), np.uint64(k2),
          np.uint64(k1) ^ np.uint64(k2) ^ np.uint64(0x1BD11BDA)]
    rot = [[13, 15, 26, 6], [17, 29, 16, 24]]
    x1 = (x1 + ks[0]) & m
    x2 = (x2 + ks[1]) & m
    for r in range(5):
        for d in rot[r % 2]:
            x1 = (x1 + x2) & m
            x2 = rotl(x2, d)
            x2 = x1 ^ x2
        x1 = (x1 + ks[(r + 1) % 3]) & m
        x2 = (x2 + ks[(r + 2) % 3] + np.uint64(r + 1)) & m
    return x1.astype(np.uint32), x2.astype(np.uint32)


def _uniform_np(seed, n):
    i = np.arange(n, dtype=np.uint64)
    hi = (i >> np.uint64(32)).astype(np.uint32)
    lo = (i & np.uint64(0xFFFFFFFF)).astype(np.uint32)
    o1, o2 = _threefry2x32_np(0, np.uint32(seed), hi, lo)
    bits = o1 ^ o2
    return (((bits >> np.uint32(9)) | np.uint32(0x3F800000)).view(np.float32)
            - np.float32(1.0))


def _mask_constants():
    noise = _uniform_np(42, _B * _L).reshape(_B, _L)
    order = np.argsort(noise, axis=1, kind="stable")
    ranks = np.argsort(order, axis=1, kind="stable")
    mask = ranks >= (_L // 4)
    ids = np.argsort(mask.reshape(_NW, _RPW), axis=1, kind="stable")
    off = (np.arange(_NW, dtype=np.int64) * _RPW)[:, None]
    kidx = (ids[:, :_NKEEP] + off).astype(np.int32).reshape(_NW, _NKC, _KC)
    midx = (ids[:, _NKEEP:] + off).astype(np.int32).reshape(_NW, _NTCA, _TCA)
    return mask, kidx, midx


_MASK_NP, _KIDX_NP, _MIDX_NP = _mask_constants()


def _proj_table_kernel(tab_ref, w_ref, b_ref, out_ref):
    out_ref[...] = (
        jnp.dot(tab_ref[...], w_ref[...], preferred_element_type=jnp.float32)
        + b_ref[...]
    )


def _build_proj_table(table, w, b):
    blk = 1024
    return pl.pallas_call(
        _proj_table_kernel,
        grid=(_V // blk,),
        in_specs=[
            pl.BlockSpec((blk, 32), lambda i: (i, 0)),
            pl.BlockSpec((32, _D), lambda i: (0, 0)),
            pl.BlockSpec((1, _D), lambda i: (0, 0)),
        ],
        out_specs=pl.BlockSpec((blk, _D), lambda i: (i, 0)),
        out_shape=jax.ShapeDtypeStruct((_V, _D), jnp.float32),
    )(table, w, b)


def _sc_token_fill(tokrep, midx_a):
    mesh = plsc.VectorSubcoreMesh(core_axis_name="c", subcore_axis_name="s")

    @functools.partial(
        pl.kernel,
        mesh=mesh,
        out_type=jax.ShapeDtypeStruct((_B * _L, _D), jnp.float32),
        scratch_types=[
            pltpu.VMEM((_TCA, _D), jnp.float32),
            pltpu.VMEM((_NTCA, _TCA), jnp.int32),
            pltpu.SemaphoreType.DMA,
        ],
    )
    def k(tok_hbm, midx_hbm, out_hbm, tokbuf, midx_v, st):
        wid = lax.axis_index("s") * 2 + lax.axis_index("c")
        pltpu.sync_copy(tok_hbm, tokbuf)
        pltpu.sync_copy(midx_hbm.at[wid], midx_v)
        th = []
        for j in range(_NTCA):
            th.append(pltpu.async_copy(tokbuf, out_hbm.at[midx_v.at[j]], st))
            if j >= 8:
                th[j - 8].wait()
        for j in range(max(0, _NTCA - 8), _NTCA):
            th[j].wait()

    return k(tokrep, midx_a)


def _sc_kept_scatter(projT, x_flat, kidx_g, out_ref):
    mesh = plsc.VectorSubcoreMesh(core_axis_name="c", subcore_axis_name="s")

    @functools.partial(
        pl.kernel,
        mesh=mesh,
        scratch_types=[
            [pltpu.VMEM((_KC, _D), jnp.float32) for _ in range(_NB)],
            [pltpu.VMEM((_KC,), jnp.int32) for _ in range(_NB)],
            pltpu.VMEM((_NKC, _KC), jnp.int32),
            [pltpu.SemaphoreType.DMA for _ in range(_NB)],
            [pltpu.SemaphoreType.DMA for _ in range(_NB)],
            [pltpu.SemaphoreType.DMA for _ in range(_NB)],
        ],
    )
    def k(tab_hbm, x_hbm, kg_hbm, out_hbm, rows, ivs, kg_v, isem, gsem, ssem):
        wid = lax.axis_index("s") * 2 + lax.axis_index("c")
        pltpu.sync_copy(kg_hbm.at[wid], kg_v)

        def idx_gather(i):
            return pltpu.async_copy(
                x_hbm.at[kg_v.at[i]], ivs[i % _NB], isem[i % _NB])

        def row_gather(i):
            return pltpu.async_copy(
                tab_hbm.at[ivs[i % _NB]], rows[i % _NB], gsem[i % _NB])

        ih = {i: idx_gather(i) for i in range(_NB)}
        gh = {}
        sh = {}
        for i in range(2):
            ih[i].wait()
            gh[i] = row_gather(i)
        for i in range(_NKC):
            gh[i].wait()
            sh[i] = pltpu.async_copy(
                rows[i % _NB], out_hbm.at[kg_v.at[i]], ssem[i % _NB])
            if i + _NB < _NKC:
                ih[i + _NB] = idx_gather(i + _NB)
            j = i + 2
            if j < _NKC:
                if j >= _NB:
                    sh[j - _NB].wait()
                ih[j].wait()
                gh[j] = row_gather(j)
        for i in range(_NKC - _NB, _NKC):
            sh[i].wait()

    return k(projT, x_flat, kidx_g, out_ref)


def kernel(x, patch_embed_weight, proj_w, proj_b, mask_token):
    mask = jnp.asarray(_MASK_NP)
    midx = jnp.asarray(_MIDX_NP)
    kidx_g = jnp.asarray(_KIDX_NP)

    x_flat = x.reshape(-1).astype(jnp.int32)
    tokrep = jnp.broadcast_to(
        mask_token.reshape(1, _D).astype(jnp.float32), (_TCA, _D))

    filled = _sc_token_fill(tokrep, midx)
    projT = _build_proj_table(
        patch_embed_weight.astype(jnp.float32),
        proj_w.astype(jnp.float32),
        proj_b.reshape(1, _D).astype(jnp.float32),
    )
    out_ref = jax.new_ref(filled)
    _sc_kept_scatter(projT, x_flat, kidx_g, out_ref)
    return out_ref[...].reshape(_B, _L, _D), mask


# speedup vs baseline: 1.0619x; 1.0039x over previous
import functools

import jax
import jax.numpy as jnp
import numpy as np
from jax import lax
from jax.experimental import pallas as pl
from jax.experimental.pallas import tpu as pltpu
from jax.experimental.pallas import tpu_sc as plsc

_L = 1024
_B = 64
_D = 768
_V = 8192
_NW = 32
_RPW = (_B * _L) // _NW
_NKEEP = _RPW // 4
_NMASK = _RPW - _NKEEP
_KC = 32
_NKC = _NKEEP // _KC
_NB = 4
_TCA = 16
_NTCA = _NMASK // _TCA


def _threefry2x32_np(k1, k2, x1, x2):
    m = np.uint64(0xFFFFFFFF)

    def rotl(x, d):
        return ((x << np.uint64(d)) | (x >> np.uint64(32 - d))) & m

    x1 = x1.astype(np.uint64)
    x2 = x2.astype(np.uint64)
    ks = [np.uint64(k1), np.uint64(k2),
          np.uint64(k1) ^ np.uint64(k2) ^ np.uint64(0x1BD11BDA)]
    rot = [[13, 15, 26, 6], [17, 29, 16, 24]]
    x1 = (x1 + ks[0]) & m
    x2 = (x2 + ks[1]) & m
    for r in range(5):
        for d in rot[r % 2]:
            x1 = (x1 + x2) & m
            x2 = rotl(x2, d)
            x2 = x1 ^ x2
        x1 = (x1 + ks[(r + 1) % 3]) & m
        x2 = (x2 + ks[(r + 2) % 3] + np.uint64(r + 1)) & m
    return x1.astype(np.uint32), x2.astype(np.uint32)


def _uniform_np(seed, n):
    i = np.arange(n, dtype=np.uint64)
    hi = (i >> np.uint64(32)).astype(np.uint32)
    lo = (i & np.uint64(0xFFFFFFFF)).astype(np.uint32)
    o1, o2 = _threefry2x32_np(0, np.uint32(seed), hi, lo)
    bits = o1 ^ o2
    return (((bits >> np.uint32(9)) | np.uint32(0x3F800000)).view(np.float32)
            - np.float32(1.0))


def _mask_constants():
    noise = _uniform_np(42, _B * _L).reshape(_B, _L)
    order = np.argsort(noise, axis=1, kind="stable")
    ranks = np.argsort(order, axis=1, kind="stable")
    mask = ranks >= (_L // 4)
    ids = np.argsort(mask.reshape(_NW, _RPW), axis=1, kind="stable")
    off = (np.arange(_NW, dtype=np.int64) * _RPW)[:, None]
    kidx = (ids[:, :_NKEEP] + off).astype(np.int32).reshape(_NW, _NKC, _KC)
    midx = (ids[:, _NKEEP:] + off).astype(np.int32).reshape(_NW, _NTCA, _TCA)
    return mask, kidx, midx


_MASK_NP, _KIDX_NP, _MIDX_NP = _mask_constants()


def _proj_table_kernel(tab_ref, w_ref, b_ref, out_ref):
    out_ref[...] = (
        jnp.dot(tab_ref[...], w_ref[...], preferred_element_type=jnp.float32)
        + b_ref[...]
    )


def _build_proj_table(table, w, b):
    blk = 1024
    return pl.pallas_call(
        _proj_table_kernel,
        grid=(_V // blk,),
        in_specs=[
            pl.BlockSpec((blk, 32), lambda i: (i, 0)),
            pl.BlockSpec((32, _D), lambda i: (0, 0)),
            pl.BlockSpec((1, _D), lambda i: (0, 0)),
        ],
        out_specs=pl.BlockSpec((blk, _D), lambda i: (i, 0)),
        out_shape=jax.ShapeDtypeStruct((_V, _D), jnp.float32),
    )(table, w, b)


def _sc_token_fill(tokrep, midx_a):
    mesh = plsc.VectorSubcoreMesh(core_axis_name="c", subcore_axis_name="s")

    @functools.partial(
        pl.kernel,
        mesh=mesh,
        out_type=jax.ShapeDtypeStruct((_B * _L, _D), jnp.float32),
        scratch_types=[
            pltpu.VMEM((_TCA, _D), jnp.float32),
            pltpu.VMEM((_NTCA, _TCA), jnp.int32),
            pltpu.SemaphoreType.DMA,
        ],
    )
    def k(tok_hbm, midx_hbm, out_hbm, tokbuf, midx_v, st):
        wid = lax.axis_index("s") * 2 + lax.axis_index("c")
        pltpu.sync_copy(tok_hbm, tokbuf)
        pltpu.sync_copy(midx_hbm.at[wid], midx_v)
        th = []
        for j in range(_NTCA):
            th.append(pltpu.async_copy(tokbuf, out_hbm.at[midx_v.at[j]], st))
            if j >= 16:
                th[j - 16].wait()
        for j in range(max(0, _NTCA - 16), _NTCA):
            th[j].wait()

    return k(tokrep, midx_a)


def _sc_kept_scatter(projT, x_flat, kidx_g, out_ref):
    mesh = plsc.VectorSubcoreMesh(core_axis_name="c", subcore_axis_name="s")

    @functools.partial(
        pl.kernel,
        mesh=mesh,
        scratch_types=[
            [pltpu.VMEM((_KC, _D), jnp.float32) for _ in range(_NB)],
            [pltpu.VMEM((_KC,), jnp.int32) for _ in range(_NB)],
            pltpu.VMEM((_NKC, _KC), jnp.int32),
            [pltpu.SemaphoreType.DMA for _ in range(_NB)],
            [pltpu.SemaphoreType.DMA for _ in range(_NB)],
            [pltpu.SemaphoreType.DMA for _ in range(_NB)],
        ],
    )
    def k(tab_hbm, x_hbm, kg_hbm, out_hbm, rows, ivs, kg_v, isem, gsem, ssem):
        wid = lax.axis_index("s") * 2 + lax.axis_index("c")
        pltpu.sync_copy(kg_hbm.at[wid], kg_v)

        def idx_gather(i):
            return pltpu.async_copy(
                x_hbm.at[kg_v.at[i]], ivs[i % _NB], isem[i % _NB])

        def row_gather(i):
            return pltpu.async_copy(
                tab_hbm.at[ivs[i % _NB]], rows[i % _NB], gsem[i % _NB])

        ih = {i: idx_gather(i) for i in range(_NB)}
        gh = {}
        sh = {}
        for i in range(2):
            ih[i].wait()
            gh[i] = row_gather(i)
        for i in range(_NKC):
            gh[i].wait()
            sh[i] = pltpu.async_copy(
                rows[i % _NB], out_hbm.at[kg_v.at[i]], ssem[i % _NB])
            if i + _NB < _NKC:
                ih[i + _NB] = idx_gather(i + _NB)
            j = i + 2
            if j < _NKC:
                if j >= _NB:
                    sh[j - _NB].wait()
                ih[j].wait()
                gh[j] = row_gather(j)
        for i in range(_NKC - _NB, _NKC):
            sh[i].wait()

    return k(projT, x_flat, kidx_g, out_ref)


def kernel(x, patch_embed_weight, proj_w, proj_b, mask_token):
    mask = jnp.asarray(_MASK_NP)
    midx = jnp.asarray(_MIDX_NP)
    kidx_g = jnp.asarray(_KIDX_NP)

    x_flat = x.reshape(-1).astype(jnp.int32)
    tokrep = jnp.broadcast_to(
        mask_token.reshape(1, _D).astype(jnp.float32), (_TCA, _D))

    filled = _sc_token_fill(tokrep, midx)
    projT = _build_proj_table(
        patch_embed_weight.astype(jnp.float32),
        proj_w.astype(jnp.float32),
        proj_b.reshape(1, _D).astype(jnp.float32),
    )
    out_ref = jax.new_ref(filled)
    _sc_kept_scatter(projT, x_flat, kidx_g, out_ref)
    return out_ref[...].reshape(_B, _L, _D), mask
